# baseline (speedup 1.0000x reference)
import functools

import numpy as np
import jax
import jax.numpy as jnp
from jax import lax
from jax.experimental import pallas as pl
from jax.experimental.pallas import tpu as pltpu
from jax.experimental.pallas import tpu_sc as plsc

N_NODES = 10000
E_EDGES = 320000
IN_DIM = 128
HEADS = 8
OUT_DIM = 16
FEAT_DIM = HEADS * OUT_DIM
ROW = FEAT_DIM + OUT_DIM
NEG_SLOPE = 0.2

NUM_CORES = 2
NUM_SUBCORES = 16
NUM_TILES = NUM_CORES * NUM_SUBCORES
CHUNK = 80
LANES = 16
ACC_ROWS = 10112

ROW_BLOCK = 1000


def _prep_body(x_ref, wt_ref, aa_ref, ab_ref, feat_ref, sa_ref, sb_ref):
    feat = jnp.dot(x_ref[...], wt_ref[...], preferred_element_type=jnp.float32)
    feat_ref[...] = feat.astype(jnp.bfloat16)
    sa_ref[...] = jnp.dot(feat, aa_ref[...], preferred_element_type=jnp.float32)
    sb_ref[...] = jnp.dot(feat, ab_ref[...], preferred_element_type=jnp.float32)


def _prep(x, w_t, acat_a, acat_b):
    n = x.shape[0]
    grid = n // ROW_BLOCK
    return pl.pallas_call(
        _prep_body,
        grid=(grid,),
        in_specs=[
            pl.BlockSpec((ROW_BLOCK, IN_DIM), lambda i: (i, 0)),
            pl.BlockSpec((IN_DIM, FEAT_DIM), lambda i: (0, 0)),
            pl.BlockSpec((FEAT_DIM, 2 * HEADS), lambda i: (0, 0)),
            pl.BlockSpec((FEAT_DIM, 2 * HEADS), lambda i: (0, 0)),
        ],
        out_specs=[
            pl.BlockSpec((ROW_BLOCK, FEAT_DIM), lambda i: (i, 0)),
            pl.BlockSpec((ROW_BLOCK, 2 * HEADS), lambda i: (i, 0)),
            pl.BlockSpec((ROW_BLOCK, 2 * HEADS), lambda i: (i, 0)),
        ],
        out_shape=[
            jax.ShapeDtypeStruct((n, FEAT_DIM), jnp.bfloat16),
            jax.ShapeDtypeStruct((n, 2 * HEADS), jnp.float32),
            jax.ShapeDtypeStruct((n, 2 * HEADS), jnp.float32),
        ],
    )(x, w_t, acat_a, acat_b)


_GDN = lax.GatherDimensionNumbers(offset_dims=(), collapsed_slice_dims=(0,),
                                  start_index_map=(0,))


def _shuf(vec, idx):
    return lax.gather(vec, idx[:, None], _GDN, (1,),
                      mode=lax.GatherScatterMode.PROMISE_IN_BOUNDS)


def _edge_body(feat_hbm, sa_hbm, sb_hbm, src_hbm, dst_hbm, out_hbm,
               srcv0, dstv0, ssv0, dsv0, fv0,
               srcv1, dstv1, ssv1, dsv1, fv1,
               dsc0, dsc1, wv, acc, sem0, sem1, semsc):
    cid = lax.axis_index("c")
    sid = lax.axis_index("s")
    wid = cid * NUM_SUBCORES + sid

    rows_per_tile = ACC_ROWS // NUM_SUBCORES

    def zrow(i, c):
        for j in range(ROW // LANES):
            wv[i, pl.ds(j * LANES, LANES)] = jnp.zeros((LANES,), jnp.float32)
        return c
    lax.fori_loop(0, CHUNK, zrow, 0)
    for r in range(rows_per_tile // CHUNK):
        pltpu.sync_copy(wv, acc.at[pl.ds(sid * rows_per_tile + r * CHUNK, CHUNK)])
    tail0 = rows_per_tile % CHUNK
    if tail0:
        pltpu.sync_copy(
            wv.at[pl.ds(0, tail0)],
            acc.at[pl.ds(sid * rows_per_tile + rows_per_tile - tail0, tail0)])
    plsc.subcore_barrier()

    per_tile = E_EDGES // NUM_TILES
    n_chunks = per_tile // CHUNK
    base_t = wid * per_tile

    lane_mask = lax.iota(jnp.int32, LANES) < HEADS
    lanes = lax.iota(jnp.int32, LANES)
    x4, x2, x1 = lanes ^ 4, lanes ^ 2, lanes ^ 1
    bidx = [jnp.full((LANES,), h, jnp.int32) for h in range(HEADS)]
    himask = jnp.full((LANES,), -65536, jnp.int32)
    zoffs = jnp.where(lane_mask, 0.0, 1.0).astype(jnp.float32)

    def fire_idx(i, srcv, dstv, semidx):
        base = base_t + i * CHUNK
        pltpu.async_copy(src_hbm.at[pl.ds(base, CHUNK)], srcv, semidx)
        pltpu.async_copy(dst_hbm.at[pl.ds(base, CHUNK)], dstv, semidx)

    def wait_idx(i, srcv, dstv, semidx):
        base = base_t + i * CHUNK
        pltpu.make_async_copy(src_hbm.at[pl.ds(base, CHUNK)], srcv, semidx).wait()
        pltpu.make_async_copy(dst_hbm.at[pl.ds(base, CHUNK)], dstv, semidx).wait()

    def fire_gathers(srcv, dstv, ssv, dsv, fv, sem):
        pltpu.async_copy(sa_hbm.at[srcv], ssv, sem)
        pltpu.async_copy(sb_hbm.at[dstv], dsv, sem)
        pltpu.async_copy(feat_hbm.at[srcv], fv, sem)

    def wait_gathers(srcv, dstv, ssv, dsv, fv, sem):
        pltpu.make_async_copy(sa_hbm.at[srcv], ssv, sem).wait()
        pltpu.make_async_copy(sb_hbm.at[dstv], dsv, sem).wait()
        pltpu.make_async_copy(feat_hbm.at[srcv], fv, sem).wait()

    def compute(srcv, dstv, ssv, dsv, fv):
        @plsc.parallel_loop(0, CHUNK, 1, unroll=2)
        def _(e):
            u = ssv[e, pl.ds(0, LANES)] + dsv[e, pl.ds(0, LANES)]
            lr = jnp.maximum(u, NEG_SLOPE * u)
            ex = jnp.where(lane_mask, jnp.exp(lr), 0.0)
            s = ex + _shuf(ex, x4)
            s = s + _shuf(s, x2)
            s = s + _shuf(s, x1) + zoffs
            pvec = ex / s
            wv[e, pl.ds(FEAT_DIM, LANES)] = pvec
            for k in range(HEADS // 2):
                fp = fv[e, pl.ds(LANES * k, LANES)]
                fa = lax.bitcast_convert_type(fp << 16, jnp.float32)
                fb = lax.bitcast_convert_type(fp & himask, jnp.float32)
                wv[e, pl.ds(32 * k, LANES)] = fa * _shuf(pvec, bidx[2 * k])
                wv[e, pl.ds(32 * k + LANES, LANES)] = fb * _shuf(pvec, bidx[2 * k + 1])

    def copy_idx(dstv, dsc):
        for j in range(CHUNK // LANES):
            dsc[pl.ds(j * LANES, LANES)] = dstv[pl.ds(j * LANES, LANES)]

    def fire_scatter(dsc):
        pltpu.async_copy(wv, acc.at[dsc], semsc, add=True)

    def wait_scatter(dsc):
        pltpu.make_async_copy(wv, acc.at[dsc], semsc).wait()

    b0 = (srcv0, dstv0, ssv0, dsv0, fv0)
    b1 = (srcv1, dstv1, ssv1, dsv1, fv1)

    fire_idx(0, srcv0, dstv0, sem0)
    wait_idx(0, srcv0, dstv0, sem0)
    fire_gathers(*b0, sem0)
    fire_idx(1, srcv1, dstv1, sem1)
    wait_gathers(*b0, sem0)
    copy_idx(dstv0, dsc0)
    fire_idx(2, srcv0, dstv0, sem0)
    wait_idx(1, srcv1, dstv1, sem1)
    fire_gathers(*b1, sem1)
    compute(*b0)
    fire_scatter(dsc0)

    def half(i, cur, nxt, dsc_cur, dsc_prev, sem_cur, sem_nxt):
        wait_idx(i + 1, nxt[0], nxt[1], sem_nxt)
        fire_gathers(*nxt, sem_nxt)
        wait_gathers(*cur, sem_cur)
        copy_idx(cur[1], dsc_cur)
        fire_idx(i + 2, cur[0], cur[1], sem_cur)
        wait_scatter(dsc_prev)
        compute(*cur)
        fire_scatter(dsc_cur)

    def pair_body(k, c):
        i = 2 * k + 1
        half(i, b1, b0, dsc1, dsc0, sem1, sem0)
        half(i + 1, b0, b1, dsc0, dsc1, sem0, sem1)
        return c

    lax.fori_loop(0, (n_chunks - 3) // 2, pair_body, 0)
    wait_idx(n_chunks - 1, srcv0, dstv0, sem0)
    fire_gathers(*b0, sem0)
    wait_gathers(*b1, sem1)
    copy_idx(dstv1, dsc1)
    wait_scatter(dsc0)
    compute(*b1)
    fire_scatter(dsc1)
    wait_gathers(*b0, sem0)
    copy_idx(dstv0, dsc0)
    wait_scatter(dsc1)
    compute(*b0)
    pltpu.sync_copy(wv, acc.at[dsc0], add=True)
    plsc.subcore_barrier()

    pltpu.sync_copy(acc.at[pl.ds(sid * rows_per_tile, rows_per_tile)],
                    out_hbm.at[cid].at[pl.ds(sid * rows_per_tile, rows_per_tile)])


def _edge(feat, scores_a, scores_b, src, dst):
    mesh = plsc.VectorSubcoreMesh(core_axis_name="c", subcore_axis_name="s",
                                  num_cores=NUM_CORES, num_subcores=NUM_SUBCORES)
    zr = 128
    call = pl.kernel(
        _edge_body,
        out_type=jax.ShapeDtypeStruct((NUM_CORES, ACC_ROWS, ROW), jnp.float32),
        mesh=mesh,
        scratch_types=[
            pltpu.VMEM((CHUNK,), jnp.int32),
            pltpu.VMEM((CHUNK,), jnp.int32),
            pltpu.VMEM((CHUNK, 2 * HEADS), jnp.float32),
            pltpu.VMEM((CHUNK, 2 * HEADS), jnp.float32),
            pltpu.VMEM((CHUNK, FEAT_DIM // 2), jnp.int32),
            pltpu.VMEM((CHUNK,), jnp.int32),
            pltpu.VMEM((CHUNK,), jnp.int32),
            pltpu.VMEM((CHUNK, 2 * HEADS), jnp.float32),
            pltpu.VMEM((CHUNK, 2 * HEADS), jnp.float32),
            pltpu.VMEM((CHUNK, FEAT_DIM // 2), jnp.int32),
            pltpu.VMEM((CHUNK,), jnp.int32),
            pltpu.VMEM((CHUNK,), jnp.int32),
            pltpu.VMEM((CHUNK, ROW), jnp.float32),
            pltpu.VMEM_SHARED((ACC_ROWS, ROW), jnp.float32),
            pltpu.SemaphoreType.DMA,
            pltpu.SemaphoreType.DMA,
            pltpu.SemaphoreType.DMA,
        ],
        compiler_params=pltpu.CompilerParams(use_tc_tiling_on_sc=False),
    )
    return call(feat, scores_a, scores_b, src, dst)


def _finish_body(p0_ref, p1_ref, k_ref, b_ref, o_ref):
    s = p0_ref[...] + p1_ref[...]
    num = s[:, :FEAT_DIM]
    den = s[:, FEAT_DIM:]
    den_exp = jnp.dot(den, k_ref[...], preferred_element_type=jnp.float32)
    o_ref[...] = num / (den_exp + 1e-16) + b_ref[...]


def _finish(p0, p1, kmat, bias_row):
    n = N_NODES
    grid = n // ROW_BLOCK
    return pl.pallas_call(
        _finish_body,
        grid=(grid,),
        in_specs=[
            pl.BlockSpec((ROW_BLOCK, ROW), lambda i: (i, 0)),
            pl.BlockSpec((ROW_BLOCK, ROW), lambda i: (i, 0)),
            pl.BlockSpec((OUT_DIM, FEAT_DIM), lambda i: (0, 0)),
            pl.BlockSpec((1, FEAT_DIM), lambda i: (0, 0)),
        ],
        out_specs=pl.BlockSpec((ROW_BLOCK, FEAT_DIM), lambda i: (i, 0)),
        out_shape=jax.ShapeDtypeStruct((n, FEAT_DIM), jnp.float32),
    )(p0, p1, kmat, bias_row)


def kernel(x, edge_index, W, att_source, att_target, bias):
    eye = jnp.eye(HEADS, dtype=jnp.float32)
    a1 = (eye[:, None, :] * att_source[0][:, :, None]).reshape(FEAT_DIM, HEADS)
    a2 = (eye[:, None, :] * att_target[0][:, :, None]).reshape(FEAT_DIM, HEADS)
    acat_a = jnp.concatenate([a1, a2], axis=1)
    acat_b = jnp.concatenate([a2, a1], axis=1)

    oldcol = np.empty(FEAT_DIM, np.int32)
    for k in range(HEADS // 2):
        for j in range(OUT_DIM):
            oldcol[32 * k + 2 * j] = 32 * k + j
            oldcol[32 * k + 2 * j + 1] = 32 * k + OUT_DIM + j
    oldcol = jnp.asarray(oldcol)

    feat, scores_a, scores_b = _prep(x, W.T[:, oldcol],
                                     acat_a[oldcol, :], acat_b[oldcol, :])
    feat_i32 = lax.bitcast_convert_type(
        feat.reshape(N_NODES, FEAT_DIM // 2, 2), jnp.int32)

    partials = _edge(feat_i32, scores_a, scores_b, edge_index[0], edge_index[1])

    kmat = np.zeros((OUT_DIM, FEAT_DIM), dtype=np.float32)
    for h in range(HEADS):
        kmat[h, h * OUT_DIM:(h + 1) * OUT_DIM] = 1.0
    kmat = jnp.asarray(kmat)

    return _finish(partials[0], partials[1], kmat, bias.reshape(1, FEAT_DIM))


# speedup vs baseline: 1.1826x; 1.1826x over previous
import functools

import numpy as np
import jax
import jax.numpy as jnp
from jax import lax
from jax.experimental import pallas as pl
from jax.experimental.pallas import tpu as pltpu
from jax.experimental.pallas import tpu_sc as plsc

N_NODES = 10000
E_EDGES = 320000
IN_DIM = 128
HEADS = 8
OUT_DIM = 16
FEAT_DIM = HEADS * OUT_DIM
ROW = FEAT_DIM + OUT_DIM
NEG_SLOPE = 0.2

NUM_CORES = 2
NUM_SUBCORES = 16
NUM_TILES = NUM_CORES * NUM_SUBCORES
CHUNK = 80
LANES = 16
ACC_ROWS = 10112

ROW_BLOCK = 1000


def _prep_body(x_ref, wt_ref, aa_ref, ab_ref, feat_ref, sa_ref, sb_ref):
    feat = jnp.dot(x_ref[...], wt_ref[...], preferred_element_type=jnp.float32)
    feat_ref[...] = feat
    sa_ref[...] = jnp.dot(feat, aa_ref[...], preferred_element_type=jnp.float32)
    sb_ref[...] = jnp.dot(feat, ab_ref[...], preferred_element_type=jnp.float32)


def _prep(x, w_t, acat_a, acat_b):
    n = x.shape[0]
    grid = n // ROW_BLOCK
    return pl.pallas_call(
        _prep_body,
        grid=(grid,),
        in_specs=[
            pl.BlockSpec((ROW_BLOCK, IN_DIM), lambda i: (i, 0)),
            pl.BlockSpec((IN_DIM, FEAT_DIM), lambda i: (0, 0)),
            pl.BlockSpec((FEAT_DIM, 2 * HEADS), lambda i: (0, 0)),
            pl.BlockSpec((FEAT_DIM, 2 * HEADS), lambda i: (0, 0)),
        ],
        out_specs=[
            pl.BlockSpec((ROW_BLOCK, FEAT_DIM), lambda i: (i, 0)),
            pl.BlockSpec((ROW_BLOCK, 2 * HEADS), lambda i: (i, 0)),
            pl.BlockSpec((ROW_BLOCK, 2 * HEADS), lambda i: (i, 0)),
        ],
        out_shape=[
            jax.ShapeDtypeStruct((n, FEAT_DIM), jnp.float32),
            jax.ShapeDtypeStruct((n, 2 * HEADS), jnp.float32),
            jax.ShapeDtypeStruct((n, 2 * HEADS), jnp.float32),
        ],
    )(x, w_t, acat_a, acat_b)


_GDN = lax.GatherDimensionNumbers(offset_dims=(), collapsed_slice_dims=(0,),
                                  start_index_map=(0,))


def _shuf(vec, idx):
    return lax.gather(vec, idx[:, None], _GDN, (1,),
                      mode=lax.GatherScatterMode.PROMISE_IN_BOUNDS)


def _edge_body(feat_hbm, sa_hbm, sb_hbm, src_hbm, dst_hbm, out_hbm,
               srcv0, dstv0, ssv0, dsv0, fv0,
               srcv1, dstv1, ssv1, dsv1, fv1,
               dsc0, dsc1, wv, acc, sem0, sem1, semsc):
    cid = lax.axis_index("c")
    sid = lax.axis_index("s")
    wid = cid * NUM_SUBCORES + sid

    rows_per_tile = ACC_ROWS // NUM_SUBCORES

    def zrow(i, c):
        for j in range(ROW // LANES):
            wv[i, pl.ds(j * LANES, LANES)] = jnp.zeros((LANES,), jnp.float32)
        return c
    lax.fori_loop(0, CHUNK, zrow, 0)
    for r in range(rows_per_tile // CHUNK):
        pltpu.sync_copy(wv, acc.at[pl.ds(sid * rows_per_tile + r * CHUNK, CHUNK)])
    tail0 = rows_per_tile % CHUNK
    if tail0:
        pltpu.sync_copy(
            wv.at[pl.ds(0, tail0)],
            acc.at[pl.ds(sid * rows_per_tile + rows_per_tile - tail0, tail0)])
    plsc.subcore_barrier()

    per_tile = E_EDGES // NUM_TILES
    n_chunks = per_tile // CHUNK
    base_t = wid * per_tile

    lanes = lax.iota(jnp.int32, LANES)
    x4, x2, x1 = lanes ^ 4, lanes ^ 2, lanes ^ 1
    bidx = [jnp.full((LANES,), h, jnp.int32) for h in range(HEADS)]

    def fire_idx(i, srcv, dstv, semidx):
        base = base_t + i * CHUNK
        pltpu.async_copy(src_hbm.at[pl.ds(base, CHUNK)], srcv, semidx)
        pltpu.async_copy(dst_hbm.at[pl.ds(base, CHUNK)], dstv, semidx)

    def wait_idx(i, srcv, dstv, semidx):
        base = base_t + i * CHUNK
        pltpu.make_async_copy(src_hbm.at[pl.ds(base, CHUNK)], srcv, semidx).wait()
        pltpu.make_async_copy(dst_hbm.at[pl.ds(base, CHUNK)], dstv, semidx).wait()

    def fire_gathers(srcv, dstv, ssv, dsv, fv, sem):
        pltpu.async_copy(sa_hbm.at[srcv], ssv, sem)
        pltpu.async_copy(sb_hbm.at[dstv], dsv, sem)
        pltpu.async_copy(feat_hbm.at[srcv], fv, sem)

    def wait_gathers(srcv, dstv, ssv, dsv, fv, sem):
        pltpu.make_async_copy(sa_hbm.at[srcv], ssv, sem).wait()
        pltpu.make_async_copy(sb_hbm.at[dstv], dsv, sem).wait()
        pltpu.make_async_copy(feat_hbm.at[srcv], fv, sem).wait()

    def compute(srcv, dstv, ssv, dsv, fv):
        @plsc.parallel_loop(0, CHUNK, 1, unroll=4)
        def _(e):
            u = ssv[e, pl.ds(0, LANES)] + dsv[e, pl.ds(0, LANES)]
            lr = jnp.maximum(u, NEG_SLOPE * u)
            ex = jnp.exp(lr)
            s = ex + _shuf(ex, x4)
            s = s + _shuf(s, x2)
            s = s + _shuf(s, x1)
            pvec = ex / s
            wv[e, pl.ds(FEAT_DIM, LANES)] = pvec
            for h in range(HEADS):
                w = _shuf(pvec, bidx[h])
                wv[e, pl.ds(h * LANES, LANES)] = fv[e, pl.ds(h * LANES, LANES)] * w

    def copy_idx(dstv, dsc):
        for j in range(CHUNK // LANES):
            dsc[pl.ds(j * LANES, LANES)] = dstv[pl.ds(j * LANES, LANES)]

    def fire_scatter(dsc):
        pltpu.async_copy(wv, acc.at[dsc], semsc, add=True)

    def wait_scatter(dsc):
        pltpu.make_async_copy(wv, acc.at[dsc], semsc).wait()

    b0 = (srcv0, dstv0, ssv0, dsv0, fv0)
    b1 = (srcv1, dstv1, ssv1, dsv1, fv1)

    fire_idx(0, srcv0, dstv0, sem0)
    wait_idx(0, srcv0, dstv0, sem0)
    fire_gathers(*b0, sem0)
    fire_idx(1, srcv1, dstv1, sem1)
    wait_gathers(*b0, sem0)
    copy_idx(dstv0, dsc0)
    fire_idx(2, srcv0, dstv0, sem0)
    wait_idx(1, srcv1, dstv1, sem1)
    fire_gathers(*b1, sem1)
    compute(*b0)
    fire_scatter(dsc0)

    def half(i, cur, nxt, dsc_cur, dsc_prev, sem_cur, sem_nxt):
        wait_idx(i + 1, nxt[0], nxt[1], sem_nxt)
        fire_gathers(*nxt, sem_nxt)
        wait_gathers(*cur, sem_cur)
        copy_idx(cur[1], dsc_cur)
        fire_idx(i + 2, cur[0], cur[1], sem_cur)
        wait_scatter(dsc_prev)
        compute(*cur)
        fire_scatter(dsc_cur)

    def pair_body(k, c):
        i = 2 * k + 1
        half(i, b1, b0, dsc1, dsc0, sem1, sem0)
        half(i + 1, b0, b1, dsc0, dsc1, sem0, sem1)
        return c

    lax.fori_loop(0, (n_chunks - 3) // 2, pair_body, 0)
    wait_idx(n_chunks - 1, srcv0, dstv0, sem0)
    fire_gathers(*b0, sem0)
    wait_gathers(*b1, sem1)
    copy_idx(dstv1, dsc1)
    wait_scatter(dsc0)
    compute(*b1)
    fire_scatter(dsc1)
    wait_gathers(*b0, sem0)
    copy_idx(dstv0, dsc0)
    wait_scatter(dsc1)
    compute(*b0)
    pltpu.sync_copy(wv, acc.at[dsc0], add=True)
    plsc.subcore_barrier()

    pltpu.sync_copy(acc.at[pl.ds(sid * rows_per_tile, rows_per_tile)],
                    out_hbm.at[cid].at[pl.ds(sid * rows_per_tile, rows_per_tile)])


def _edge(feat, scores_a, scores_b, src, dst):
    mesh = plsc.VectorSubcoreMesh(core_axis_name="c", subcore_axis_name="s",
                                  num_cores=NUM_CORES, num_subcores=NUM_SUBCORES)
    zr = 128
    call = pl.kernel(
        _edge_body,
        out_type=jax.ShapeDtypeStruct((NUM_CORES, ACC_ROWS, ROW), jnp.float32),
        mesh=mesh,
        scratch_types=[
            pltpu.VMEM((CHUNK,), jnp.int32),
            pltpu.VMEM((CHUNK,), jnp.int32),
            pltpu.VMEM((CHUNK, 2 * HEADS), jnp.float32),
            pltpu.VMEM((CHUNK, 2 * HEADS), jnp.float32),
            pltpu.VMEM((CHUNK, FEAT_DIM), jnp.float32),
            pltpu.VMEM((CHUNK,), jnp.int32),
            pltpu.VMEM((CHUNK,), jnp.int32),
            pltpu.VMEM((CHUNK, 2 * HEADS), jnp.float32),
            pltpu.VMEM((CHUNK, 2 * HEADS), jnp.float32),
            pltpu.VMEM((CHUNK, FEAT_DIM), jnp.float32),
            pltpu.VMEM((CHUNK,), jnp.int32),
            pltpu.VMEM((CHUNK,), jnp.int32),
            pltpu.VMEM((CHUNK, ROW), jnp.float32),
            pltpu.VMEM_SHARED((ACC_ROWS, ROW), jnp.float32),
            pltpu.SemaphoreType.DMA,
            pltpu.SemaphoreType.DMA,
            pltpu.SemaphoreType.DMA,
        ],
        compiler_params=pltpu.CompilerParams(use_tc_tiling_on_sc=False),
    )
    return call(feat, scores_a, scores_b, src, dst)


def _finish_body(p0_ref, p1_ref, k_ref, b_ref, o_ref):
    s = p0_ref[...] + p1_ref[...]
    num = s[:, :FEAT_DIM]
    den = s[:, FEAT_DIM:]
    den_exp = jnp.dot(den, k_ref[...], preferred_element_type=jnp.float32)
    o_ref[...] = num / (den_exp + 1e-16) + b_ref[...]


def _finish(p0, p1, kmat, bias_row):
    n = N_NODES
    grid = n // ROW_BLOCK
    return pl.pallas_call(
        _finish_body,
        grid=(grid,),
        in_specs=[
            pl.BlockSpec((ROW_BLOCK, ROW), lambda i: (i, 0)),
            pl.BlockSpec((ROW_BLOCK, ROW), lambda i: (i, 0)),
            pl.BlockSpec((OUT_DIM, FEAT_DIM), lambda i: (0, 0)),
            pl.BlockSpec((1, FEAT_DIM), lambda i: (0, 0)),
        ],
        out_specs=pl.BlockSpec((ROW_BLOCK, FEAT_DIM), lambda i: (i, 0)),
        out_shape=jax.ShapeDtypeStruct((n, FEAT_DIM), jnp.float32),
    )(p0, p1, kmat, bias_row)


def kernel(x, edge_index, W, att_source, att_target, bias):
    eye = jnp.eye(HEADS, dtype=jnp.float32)
    a1 = (eye[:, None, :] * att_source[0][:, :, None]).reshape(FEAT_DIM, HEADS)
    a2 = (eye[:, None, :] * att_target[0][:, :, None]).reshape(FEAT_DIM, HEADS)
    acat_a = jnp.concatenate([a1, a2], axis=1)
    acat_b = jnp.concatenate([a2, a1], axis=1)

    feat, scores_a, scores_b = _prep(x, W.T, acat_a, acat_b)

    partials = _edge(feat, scores_a, scores_b, edge_index[0], edge_index[1])

    kmat = np.zeros((OUT_DIM, FEAT_DIM), dtype=np.float32)
    for h in range(HEADS):
        kmat[h, h * OUT_DIM:(h + 1) * OUT_DIM] = 1.0
    kmat = jnp.asarray(kmat)

    return _finish(partials[0], partials[1], kmat, bias.reshape(1, FEAT_DIM))


# speedup vs baseline: 1.1950x; 1.0105x over previous
import functools

import numpy as np
import jax
import jax.numpy as jnp
from jax import lax
from jax.experimental import pallas as pl
from jax.experimental.pallas import tpu as pltpu
from jax.experimental.pallas import tpu_sc as plsc

N_NODES = 10000
E_EDGES = 320000
IN_DIM = 128
HEADS = 8
OUT_DIM = 16
FEAT_DIM = HEADS * OUT_DIM
ROW = FEAT_DIM + OUT_DIM
NEG_SLOPE = 0.2

NUM_CORES = 2
NUM_SUBCORES = 16
NUM_TILES = NUM_CORES * NUM_SUBCORES
CHUNK = 80
LANES = 16
ACC_ROWS = 10112

ROW_BLOCK = 1000


def _prep_body(x_ref, wt_ref, aa_ref, ab_ref, fsa_ref, sb_ref):
    feat = jnp.dot(x_ref[...], wt_ref[...], preferred_element_type=jnp.float32)
    sa = jnp.dot(feat, aa_ref[...], preferred_element_type=jnp.float32)
    fsa_ref[...] = jnp.concatenate([feat, sa], axis=1)
    sb_ref[...] = jnp.dot(feat, ab_ref[...], preferred_element_type=jnp.float32)


def _prep(x, w_t, acat_a, acat_b):
    n = x.shape[0]
    grid = n // ROW_BLOCK
    return pl.pallas_call(
        _prep_body,
        grid=(grid,),
        in_specs=[
            pl.BlockSpec((ROW_BLOCK, IN_DIM), lambda i: (i, 0)),
            pl.BlockSpec((IN_DIM, FEAT_DIM), lambda i: (0, 0)),
            pl.BlockSpec((FEAT_DIM, 2 * HEADS), lambda i: (0, 0)),
            pl.BlockSpec((FEAT_DIM, 2 * HEADS), lambda i: (0, 0)),
        ],
        out_specs=[
            pl.BlockSpec((ROW_BLOCK, ROW), lambda i: (i, 0)),
            pl.BlockSpec((ROW_BLOCK, 2 * HEADS), lambda i: (i, 0)),
        ],
        out_shape=[
            jax.ShapeDtypeStruct((n, ROW), jnp.float32),
            jax.ShapeDtypeStruct((n, 2 * HEADS), jnp.float32),
        ],
    )(x, w_t, acat_a, acat_b)


_GDN = lax.GatherDimensionNumbers(offset_dims=(), collapsed_slice_dims=(0,),
                                  start_index_map=(0,))


def _shuf(vec, idx):
    return lax.gather(vec, idx[:, None], _GDN, (1,),
                      mode=lax.GatherScatterMode.PROMISE_IN_BOUNDS)


def _edge_body(fsa_hbm, sb_hbm, ei_hbm, out_hbm,
               ij0, fsv0, dsv0, ij1, fsv1, dsv1,
               dsc0, dsc1, wv, acc, sem0, sem1, semsc):
    cid = lax.axis_index("c")
    sid = lax.axis_index("s")
    wid = cid * NUM_SUBCORES + sid

    rows_per_tile = ACC_ROWS // NUM_SUBCORES

    def zrow(i, c):
        for j in range(ROW // LANES):
            wv[i, pl.ds(j * LANES, LANES)] = jnp.zeros((LANES,), jnp.float32)
        return c
    lax.fori_loop(0, CHUNK, zrow, 0)
    for r in range(rows_per_tile // CHUNK):
        pltpu.sync_copy(wv, acc.at[pl.ds(sid * rows_per_tile + r * CHUNK, CHUNK)])
    tail0 = rows_per_tile % CHUNK
    if tail0:
        pltpu.sync_copy(
            wv.at[pl.ds(0, tail0)],
            acc.at[pl.ds(sid * rows_per_tile + rows_per_tile - tail0, tail0)])
    plsc.subcore_barrier()

    per_tile = E_EDGES // NUM_TILES
    n_chunks = per_tile // CHUNK
    base_t = wid * per_tile

    lanes = lax.iota(jnp.int32, LANES)
    x4, x2, x1 = lanes ^ 4, lanes ^ 2, lanes ^ 1
    bidx = [jnp.full((LANES,), h, jnp.int32) for h in range(HEADS)]

    def fire_idx(i, ij, semidx):
        base = base_t + i * CHUNK
        pltpu.async_copy(ei_hbm.at[:, pl.ds(base, CHUNK)], ij, semidx)

    def wait_idx(i, ij, semidx):
        base = base_t + i * CHUNK
        pltpu.make_async_copy(ei_hbm.at[:, pl.ds(base, CHUNK)], ij, semidx).wait()

    def fire_gathers(ij, fsv, dsv, sem):
        pltpu.async_copy(fsa_hbm.at[ij.at[0]], fsv, sem)
        pltpu.async_copy(sb_hbm.at[ij.at[1]], dsv, sem)

    def wait_gathers(ij, fsv, dsv, sem):
        pltpu.make_async_copy(fsa_hbm.at[ij.at[0]], fsv, sem).wait()
        pltpu.make_async_copy(sb_hbm.at[ij.at[1]], dsv, sem).wait()

    def compute(ij, fsv, dsv):
        @plsc.parallel_loop(0, CHUNK, 1, unroll=2)
        def _(e):
            u = fsv[e, pl.ds(FEAT_DIM, LANES)] + dsv[e, pl.ds(0, LANES)]
            lr = jnp.maximum(u, NEG_SLOPE * u)
            ex = jnp.exp(lr)
            s = ex + _shuf(ex, x4)
            s = s + _shuf(s, x2)
            s = s + _shuf(s, x1)
            pvec = ex / s
            wv[e, pl.ds(FEAT_DIM, LANES)] = pvec
            for h in range(HEADS):
                w = _shuf(pvec, bidx[h])
                wv[e, pl.ds(h * LANES, LANES)] = fsv[e, pl.ds(h * LANES, LANES)] * w

    def copy_idx(ij, dsc):
        for j in range(CHUNK // LANES):
            dsc[pl.ds(j * LANES, LANES)] = ij[1, pl.ds(j * LANES, LANES)]

    def fire_scatter(dsc):
        pltpu.async_copy(wv, acc.at[dsc], semsc, add=True)

    def wait_scatter(dsc):
        pltpu.make_async_copy(wv, acc.at[dsc], semsc).wait()

    b0 = (ij0, fsv0, dsv0)
    b1 = (ij1, fsv1, dsv1)

    fire_idx(0, ij0, sem0)
    wait_idx(0, ij0, sem0)
    fire_gathers(*b0, sem0)
    fire_idx(1, ij1, sem1)
    wait_gathers(*b0, sem0)
    copy_idx(ij0, dsc0)
    fire_idx(2, ij0, sem0)
    wait_idx(1, ij1, sem1)
    fire_gathers(*b1, sem1)
    compute(*b0)
    fire_scatter(dsc0)

    def half(i, cur, nxt, dsc_cur, dsc_prev, sem_cur, sem_nxt):
        wait_idx(i + 1, nxt[0], sem_nxt)
        fire_gathers(*nxt, sem_nxt)
        wait_gathers(*cur, sem_cur)
        copy_idx(cur[0], dsc_cur)
        fire_idx(i + 2, cur[0], sem_cur)
        wait_scatter(dsc_prev)
        compute(*cur)
        fire_scatter(dsc_cur)

    def pair_body(k, c):
        i = 2 * k + 1
        half(i, b1, b0, dsc1, dsc0, sem1, sem0)
        half(i + 1, b0, b1, dsc0, dsc1, sem0, sem1)
        return c

    lax.fori_loop(0, (n_chunks - 3) // 2, pair_body, 0)
    wait_idx(n_chunks - 1, ij0, sem0)
    fire_gathers(*b0, sem0)
    wait_gathers(*b1, sem1)
    copy_idx(ij1, dsc1)
    wait_scatter(dsc0)
    compute(*b1)
    fire_scatter(dsc1)
    wait_gathers(*b0, sem0)
    copy_idx(ij0, dsc0)
    wait_scatter(dsc1)
    compute(*b0)
    pltpu.sync_copy(wv, acc.at[dsc0], add=True)
    plsc.subcore_barrier()

    pltpu.sync_copy(acc.at[pl.ds(sid * rows_per_tile, rows_per_tile)],
                    out_hbm.at[cid].at[pl.ds(sid * rows_per_tile, rows_per_tile)])


def _edge(fsa, sb, ei):
    mesh = plsc.VectorSubcoreMesh(core_axis_name="c", subcore_axis_name="s",
                                  num_cores=NUM_CORES, num_subcores=NUM_SUBCORES)
    call = pl.kernel(
        _edge_body,
        out_type=jax.ShapeDtypeStruct((NUM_CORES, ACC_ROWS, ROW), jnp.float32),
        mesh=mesh,
        scratch_types=[
            pltpu.VMEM((2, CHUNK), jnp.int32),
            pltpu.VMEM((CHUNK, ROW), jnp.float32),
            pltpu.VMEM((CHUNK, 2 * HEADS), jnp.float32),
            pltpu.VMEM((2, CHUNK), jnp.int32),
            pltpu.VMEM((CHUNK, ROW), jnp.float32),
            pltpu.VMEM((CHUNK, 2 * HEADS), jnp.float32),
            pltpu.VMEM((CHUNK,), jnp.int32),
            pltpu.VMEM((CHUNK,), jnp.int32),
            pltpu.VMEM((CHUNK, ROW), jnp.float32),
            pltpu.VMEM_SHARED((ACC_ROWS, ROW), jnp.float32),
            pltpu.SemaphoreType.DMA,
            pltpu.SemaphoreType.DMA,
            pltpu.SemaphoreType.DMA,
        ],
        compiler_params=pltpu.CompilerParams(use_tc_tiling_on_sc=False),
    )
    return call(fsa, sb, ei)


def _finish_body(p0_ref, p1_ref, k_ref, b_ref, o_ref):
    s = p0_ref[...] + p1_ref[...]
    num = s[:, :FEAT_DIM]
    den = s[:, FEAT_DIM:]
    den_exp = jnp.dot(den, k_ref[...], preferred_element_type=jnp.float32)
    o_ref[...] = num / (den_exp + 1e-16) + b_ref[...]


def _finish(p0, p1, kmat, bias_row):
    n = N_NODES
    grid = n // ROW_BLOCK
    return pl.pallas_call(
        _finish_body,
        grid=(grid,),
        in_specs=[
            pl.BlockSpec((ROW_BLOCK, ROW), lambda i: (i, 0)),
            pl.BlockSpec((ROW_BLOCK, ROW), lambda i: (i, 0)),
            pl.BlockSpec((OUT_DIM, FEAT_DIM), lambda i: (0, 0)),
            pl.BlockSpec((1, FEAT_DIM), lambda i: (0, 0)),
        ],
        out_specs=pl.BlockSpec((ROW_BLOCK, FEAT_DIM), lambda i: (i, 0)),
        out_shape=jax.ShapeDtypeStruct((n, FEAT_DIM), jnp.float32),
    )(p0, p1, kmat, bias_row)


def kernel(x, edge_index, W, att_source, att_target, bias):
    eye = jnp.eye(HEADS, dtype=jnp.float32)
    a1 = (eye[:, None, :] * att_source[0][:, :, None]).reshape(FEAT_DIM, HEADS)
    a2 = (eye[:, None, :] * att_target[0][:, :, None]).reshape(FEAT_DIM, HEADS)
    acat_a = jnp.concatenate([a1, a2], axis=1)
    acat_b = jnp.concatenate([a2, a1], axis=1)

    fsa, sb = _prep(x, W.T, acat_a, acat_b)

    partials = _edge(fsa, sb, edge_index)

    kmat = np.zeros((OUT_DIM, FEAT_DIM), dtype=np.float32)
    for h in range(HEADS):
        kmat[h, h * OUT_DIM:(h + 1) * OUT_DIM] = 1.0
    kmat = jnp.asarray(kmat)

    return _finish(partials[0], partials[1], kmat, bias.reshape(1, FEAT_DIM))
